# baseline (speedup 1.0000x reference)
import functools

import jax
import jax.numpy as jnp
from jax import lax
from jax.experimental import pallas as pl
from jax.experimental.pallas import tpu as pltpu
from jax.experimental.pallas import tpu_sc as plsc

D_MODEL = 4096
N_EXPERTS = 64
TOP_K = 8
AUX_W = 0.01
N_TOK = 16384
NW = 32
TPW = N_TOK // NW


def _gate_body(x_ref, w_ref, l_ref, aux_ref, counts_ref, *, blk_t, n_blk,
               n_tokens):
    i = pl.program_id(0)
    logits = jax.lax.dot_general(
        x_ref[...], w_ref[...],
        dimension_numbers=(((1,), (1,)), ((), ())),
        preferred_element_type=jnp.float32,
    )
    l_ref[...] = logits

    lane = jax.lax.broadcasted_iota(jnp.int32, (blk_t, N_EXPERTS), 1)
    m = jnp.max(logits, axis=1, keepdims=True)
    i0 = jnp.min(jnp.where(logits == m, lane, N_EXPERTS),
                 axis=1, keepdims=True)
    partial = jnp.sum(jnp.where(lane == i0, 1.0, 0.0), axis=0, keepdims=True)

    @pl.when(i == 0)
    def _init():
        counts_ref[...] = partial

    @pl.when(i > 0)
    def _acc():
        counts_ref[...] += partial

    @pl.when(i == n_blk - 1)
    def _fin():
        freq = counts_ref[...] * (1.0 / n_tokens)
        diff = freq - (1.0 / N_EXPERTS)
        aux_ref[...] = AUX_W * N_EXPERTS * jnp.sum(diff * diff,
                                                   axis=(0, 1), keepdims=True)


def _sc_topk_body(l_hbm, gates_hbm, idx_hbm, l_v, g_v, i_v):
    wid = lax.axis_index("s") * 2 + lax.axis_index("c")
    base = wid * TPW
    pltpu.sync_copy(l_hbm.at[pl.ds(base * N_EXPERTS, TPW * N_EXPERTS)], l_v)

    lane = jax.lax.broadcasted_iota(jnp.int32, (16,), 0)
    lo8 = lane < TOP_K

    def merge(ka, va, kb, vb):
        kbr = jax.lax.rev(kb, (0,))
        vbr = jax.lax.rev(vb, (0,))
        ck = jnp.where(lo8, ka, kbr)
        cv = jnp.where(lo8, va, vbr)
        return jax.lax.sort((ck, cv), dimension=0, num_keys=1)

    def tok(t, carry):
        sks, svs = [], []
        for j in range(4):
            k = l_v[pl.ds(t * N_EXPERTS + 16 * j, 16)]
            sk, sv = jax.lax.sort((-k, lane + 16 * j), dimension=0,
                                  num_keys=1)
            sks.append(sk)
            svs.append(sv)
        k12, v12 = merge(sks[0], svs[0], sks[1], svs[1])
        k34, v34 = merge(sks[2], svs[2], sks[3], svs[3])
        kf, vf = merge(k12, v12, k34, v34)

        m = jnp.min(kf, axis=0)
        eg = jnp.where(lo8, jnp.exp(m - kf), 0.0)
        g = eg / jnp.sum(eg, axis=0)
        plsc.store_compressed(g_v.at[pl.ds(t * TOP_K, 16)], g, mask=lo8)
        plsc.store_compressed(i_v.at[pl.ds(t * TOP_K, 16)], vf, mask=lo8)
        return carry

    jax.lax.fori_loop(0, TPW, tok, 0, unroll=2)

    out0 = base * TOP_K
    pltpu.sync_copy(g_v.at[pl.ds(0, TPW * TOP_K)],
                    gates_hbm.at[pl.ds(out0, TPW * TOP_K)])
    pltpu.sync_copy(i_v.at[pl.ds(0, TPW * TOP_K)],
                    idx_hbm.at[pl.ds(out0, TPW * TOP_K)])


def kernel(x, gate_w):
    b, s, d = x.shape
    n_tokens = b * s
    blk_t = 1024
    n_blk = n_tokens // blk_t
    xf = x.reshape(n_tokens, d)

    logits, aux = pl.pallas_call(
        functools.partial(_gate_body, blk_t=blk_t, n_blk=n_blk,
                          n_tokens=n_tokens),
        grid=(n_blk,),
        in_specs=[
            pl.BlockSpec((blk_t, d), lambda i: (i, 0)),
            pl.BlockSpec((N_EXPERTS, d), lambda i: (0, 0)),
        ],
        out_specs=[
            pl.BlockSpec((blk_t, N_EXPERTS), lambda i: (i, 0)),
            pl.BlockSpec((1, 1), lambda i: (0, 0)),
        ],
        out_shape=[
            jax.ShapeDtypeStruct((n_tokens, N_EXPERTS), jnp.float32),
            jax.ShapeDtypeStruct((1, 1), jnp.float32),
        ],
        scratch_shapes=[pltpu.VMEM((1, N_EXPERTS), jnp.float32)],
    )(xf, gate_w)

    sc_topk = functools.partial(
        pl.kernel,
        mesh=plsc.VectorSubcoreMesh(core_axis_name="c", subcore_axis_name="s"),
        compiler_params=pltpu.CompilerParams(needs_layout_passes=False),
        out_type=[
            jax.ShapeDtypeStruct((n_tokens * TOP_K,), jnp.float32),
            jax.ShapeDtypeStruct((n_tokens * TOP_K,), jnp.int32),
        ],
        scratch_types=[
            pltpu.VMEM((TPW * N_EXPERTS,), jnp.float32),
            pltpu.VMEM((TPW * TOP_K + 16,), jnp.float32),
            pltpu.VMEM((TPW * TOP_K + 16,), jnp.int32),
        ],
    )(_sc_topk_body)

    gates_flat, idx_flat = sc_topk(logits.reshape(-1))

    return (gates_flat.reshape(b, s, TOP_K),
            idx_flat.reshape(b, s, TOP_K),
            aux[0, 0])


# speedup vs baseline: 1.6962x; 1.6962x over previous
import functools

import jax
import jax.numpy as jnp
from jax.experimental import pallas as pl
from jax.experimental.pallas import tpu as pltpu

D_MODEL = 4096
N_EXPERTS = 64
TOP_K = 8
AUX_W = 0.01
CHUNK = 128


def _router_body(x1_ref, x2_ref, w_ref, gates_ref, idx_ref, aux_ref,
                 counts_ref, *, blk_t, n_blk, n_tokens):
    i = pl.program_id(0)
    half = D_MODEL // 2
    lt = jax.lax.dot_general(
        w_ref[:, :half], x1_ref[...],
        dimension_numbers=(((1,), (1,)), ((), ())),
        preferred_element_type=jnp.float32,
    ) + jax.lax.dot_general(
        w_ref[:, half:], x2_ref[...],
        dimension_numbers=(((1,), (1,)), ((), ())),
        preferred_element_type=jnp.float32,
    )

    sub_iota = jax.lax.broadcasted_iota(
        jnp.int32, (N_EXPERTS, CHUNK), 0).astype(jnp.float32)

    @pl.when(i == 0)
    def _init():
        counts_ref[...] = jnp.zeros_like(counts_ref)

    for c in range(blk_t // CHUNK):
        work = jax.lax.slice(lt, (0, c * CHUNK), (N_EXPERTS, (c + 1) * CHUNK))
        vals = []
        idxs = []
        for _ in range(TOP_K):
            mj = jnp.max(work, axis=0, keepdims=True)
            ij = jnp.min(jnp.where(work == mj, sub_iota, float(N_EXPERTS)),
                         axis=0, keepdims=True)
            vals.append(mj)
            idxs.append(ij)
            work = jnp.where(sub_iota == ij, -jnp.inf, work)

        v = jnp.concatenate(vals, axis=0)
        ev = jnp.exp(v - vals[0])
        g = ev / jnp.sum(ev, axis=0, keepdims=True)
        ix = jnp.concatenate(idxs, axis=0)

        gates_ref[pl.ds(c * CHUNK, CHUNK), :] = g.T
        idx_ref[pl.ds(c * CHUNK, CHUNK), :] = ix.T.astype(jnp.int32)

        counts_ref[...] += jnp.where(sub_iota == idxs[0], 1.0, 0.0)

    @pl.when(i == n_blk - 1)
    def _fin():
        freq = jnp.sum(counts_ref[...], axis=1, keepdims=True) / n_tokens
        diff = freq - (1.0 / N_EXPERTS)
        aux_ref[...] = AUX_W * N_EXPERTS * jnp.sum(diff * diff,
                                                   axis=(0, 1), keepdims=True)


def kernel(x, gate_w):
    b, s, d = x.shape
    n_tokens = b * s
    blk_t = 1024
    n_blk = n_tokens // blk_t
    xf = x.reshape(n_tokens, d)

    gates, idx, aux = pl.pallas_call(
        functools.partial(_router_body, blk_t=blk_t, n_blk=n_blk,
                          n_tokens=n_tokens),
        grid=(n_blk,),
        in_specs=[
            pl.BlockSpec((blk_t, d // 2), lambda i: (i, 0)),
            pl.BlockSpec((blk_t, d // 2), lambda i: (i, 1)),
            pl.BlockSpec((N_EXPERTS, d), lambda i: (0, 0)),
        ],
        out_specs=[
            pl.BlockSpec((blk_t, TOP_K), lambda i: (i, 0)),
            pl.BlockSpec((blk_t, TOP_K), lambda i: (i, 0)),
            pl.BlockSpec((1, 1), lambda i: (0, 0)),
        ],
        out_shape=[
            jax.ShapeDtypeStruct((n_tokens, TOP_K), jnp.float32),
            jax.ShapeDtypeStruct((n_tokens, TOP_K), jnp.int32),
            jax.ShapeDtypeStruct((1, 1), jnp.float32),
        ],
        scratch_shapes=[pltpu.VMEM((N_EXPERTS, CHUNK), jnp.float32)],
        compiler_params=pltpu.CompilerParams(
            vmem_limit_bytes=128 * 1024 * 1024),
    )(xf, xf, gate_w)

    return (gates.reshape(b, s, TOP_K), idx.reshape(b, s, TOP_K), aux[0, 0])
